# baseline (speedup 1.0000x reference)
import functools

import jax
import jax.numpy as jnp
from jax.experimental import pallas as pl
from jax.experimental.pallas import tpu as pltpu

MAX_LEVELS = 3
_KCHUNK = 2048
_PRECISION = jax.lax.Precision.HIGHEST


def _pick(n, cands):
    for c in cands:
        if n % c == 0:
            return c
    return n


def _kchunks(n):
    offs, sizes = [], []
    o = 0
    while o < n:
        sz = min(_KCHUNK, n - o)
        offs.append(o)
        sizes.append(sz)
        o += sz
    return tuple(zip(offs, sizes))


def _body(adj_ref, x_ref, labels_ref, w_n2l_ref, b_n2l_ref, conv_w_ref,
          conv_b_ref, last_w_ref, last_b_ref,
          pred_ref, loss_ref, acc_ref,
          embed_ref, in_lin_ref, lsum_ref,
          *, bm, rb, n_nodes, n_classes, n_blocks):
    lv = pl.program_id(0)
    i = pl.program_id(1)

    @pl.when(jnp.logical_and(lv == 0, i == 0))
    def _init():
        def chunk(r, carry):
            xa = x_ref[pl.ds(r * rb, rb), :]
            il = jnp.dot(xa, w_n2l_ref[...], precision=_PRECISION)
            il = il + b_n2l_ref[...]
            in_lin_ref[pl.ds(r * rb, rb), :] = il
            embed_ref[pl.ds(r * rb, rb), :] = jnp.maximum(il, 0.0)
            return carry
        jax.lax.fori_loop(0, n_nodes // rb, chunk, 0)

    buf = lv % 2
    n2npool = None
    for off, sz in _kchunks(n_nodes):
        e_chunk = embed_ref[pl.ds(buf * n_nodes + off, sz), :]
        part = jnp.dot(adj_ref[:, off:off + sz], e_chunk,
                       precision=_PRECISION)
        n2npool = part if n2npool is None else n2npool + part

    node_linear = jnp.dot(n2npool, conv_w_ref[...], precision=_PRECISION)
    merged = node_linear + conv_b_ref[...] + in_lin_ref[pl.ds(i * bm, bm), :]
    old_rows = embed_ref[pl.ds(buf * n_nodes + i * bm, bm), :]
    new_embed = jnp.maximum(merged, 0.0) + old_rows

    @pl.when(lv < MAX_LEVELS - 1)
    def _update():
        embed_ref[pl.ds((1 - buf) * n_nodes + i * bm, bm), :] = new_embed

    @pl.when(lv == MAX_LEVELS - 1)
    def _head():
        logits = jnp.dot(new_embed, last_w_ref[...], precision=_PRECISION)
        logits = logits + last_b_ref[...]
        m = jnp.max(logits, axis=1, keepdims=True)
        shifted = logits - m
        lse = jnp.log(jnp.sum(jnp.exp(shifted), axis=1, keepdims=True))
        ls = shifted - lse
        col = jax.lax.broadcasted_iota(jnp.int32, ls.shape, 1)
        is_max = ls >= jnp.max(ls, axis=1, keepdims=True)
        pred = jnp.min(jnp.where(is_max, col, n_classes), axis=1)
        pred_ref[0, 0, :] = pred
        labels = labels_ref[0, 0, :]
        acc_ref[0, 0, :] = (pred == labels).astype(jnp.int32)
        sel = jnp.sum(jnp.where(col == labels[:, None], ls, 0.0), axis=1)
        part_loss = jnp.sum(sel)
        total = jnp.where(i == 0, part_loss, lsum_ref[0] + part_loss)
        lsum_ref[0] = total

        @pl.when(i == n_blocks - 1)
        def _emit_loss():
            loss_ref[...] = jnp.full((1, 1), -total / n_nodes, jnp.float32)


def kernel(x, adj, labels, w_n2l, b_n2l, conv_w, conv_b, last_w, last_b):
    n, f_dim = x.shape
    l_dim = conv_w.shape[0]
    n_classes = last_w.shape[1]
    bm = _pick(n, (200, 256, 128, 100, 80, 50, 40, 25, 16, 10, 8, 5, 4, 2, 1))
    rb = _pick(n, (1000, 800, 500, 400, 200, 100, 50, 25, 10, 8, 5, 4, 2, 1))
    ni = n // bm

    labels3 = labels.astype(jnp.int32).reshape(ni, 1, bm)
    b_n2l2 = b_n2l.reshape(1, l_dim)
    conv_b2 = conv_b.reshape(1, l_dim)
    last_b2 = last_b.reshape(1, n_classes)

    body = functools.partial(_body, bm=bm, rb=rb, n_nodes=n,
                             n_classes=n_classes, n_blocks=ni)
    pred3, loss2, acc3 = pl.pallas_call(
        body,
        grid=(MAX_LEVELS, ni),
        in_specs=[
            pl.BlockSpec((bm, n), lambda lv, i: (i, 0)),
            pl.BlockSpec((n, f_dim), lambda lv, i: (0, 0)),
            pl.BlockSpec((1, 1, bm), lambda lv, i: (i, 0, 0)),
            pl.BlockSpec((f_dim, l_dim), lambda lv, i: (0, 0)),
            pl.BlockSpec((1, l_dim), lambda lv, i: (0, 0)),
            pl.BlockSpec((l_dim, l_dim), lambda lv, i: (0, 0)),
            pl.BlockSpec((1, l_dim), lambda lv, i: (0, 0)),
            pl.BlockSpec((l_dim, n_classes), lambda lv, i: (0, 0)),
            pl.BlockSpec((1, n_classes), lambda lv, i: (0, 0)),
        ],
        out_specs=(
            pl.BlockSpec((1, 1, bm), lambda lv, i: (i, 0, 0)),
            pl.BlockSpec((1, 1), lambda lv, i: (0, 0)),
            pl.BlockSpec((1, 1, bm), lambda lv, i: (i, 0, 0)),
        ),
        out_shape=(
            jax.ShapeDtypeStruct((ni, 1, bm), jnp.int32),
            jax.ShapeDtypeStruct((1, 1), jnp.float32),
            jax.ShapeDtypeStruct((ni, 1, bm), jnp.int32),
        ),
        scratch_shapes=[
            pltpu.VMEM((2 * n, l_dim), jnp.float32),
            pltpu.VMEM((n, l_dim), jnp.float32),
            pltpu.SMEM((1,), jnp.float32),
        ],
        compiler_params=pltpu.CompilerParams(
            dimension_semantics=("arbitrary", "arbitrary"),
        ),
    )(adj, x, labels3, w_n2l, b_n2l2, conv_w, conv_b2, last_w, last_b2)

    pred = pred3.reshape(n, 1)
    loss = loss2[0, 0]
    acc = acc3.reshape(n) != 0
    return pred, loss, acc


# speedup vs baseline: 2.6606x; 2.6606x over previous
import functools

import jax
import jax.numpy as jnp
from jax.experimental import pallas as pl
from jax.experimental.pallas import tpu as pltpu

MAX_LEVELS = 3
_KCHUNK = 2048
_PRECISION = jax.lax.Precision.HIGHEST
_ADJ_PRECISION = jax.lax.Precision.DEFAULT


def _pick(n, cands):
    for c in cands:
        if n % c == 0:
            return c
    return n


def _kchunks(n):
    offs, sizes = [], []
    o = 0
    while o < n:
        sz = min(_KCHUNK, n - o)
        offs.append(o)
        sizes.append(sz)
        o += sz
    return tuple(zip(offs, sizes))


def _body(adj_ref, x_ref, labels_ref, w_n2l_ref, b_n2l_ref, conv_w_ref,
          conv_b_ref, last_w_ref, last_b_ref,
          pred_ref, loss_ref, acc_ref,
          embed_ref, in_lin_ref, lsum_ref,
          *, bm, rb, n_nodes, n_classes, n_blocks):
    lv = pl.program_id(0)
    i = pl.program_id(1)

    @pl.when(jnp.logical_and(lv == 0, i == 0))
    def _init():
        def chunk(r, carry):
            xa = x_ref[pl.ds(r * rb, rb), :]
            il = jnp.dot(xa, w_n2l_ref[...], precision=_PRECISION)
            il = il + b_n2l_ref[...]
            in_lin_ref[pl.ds(r * rb, rb), :] = il
            embed_ref[pl.ds(r * rb, rb), :] = jnp.maximum(il, 0.0)
            return carry
        jax.lax.fori_loop(0, n_nodes // rb, chunk, 0)

    buf = lv % 2
    n2npool = None
    for off, sz in _kchunks(n_nodes):
        e_chunk = embed_ref[pl.ds(buf * n_nodes + off, sz), :]
        part = jnp.dot(adj_ref[:, off:off + sz], e_chunk,
                       precision=_ADJ_PRECISION)
        n2npool = part if n2npool is None else n2npool + part

    node_linear = jnp.dot(n2npool, conv_w_ref[...], precision=_PRECISION)
    merged = node_linear + conv_b_ref[...] + in_lin_ref[pl.ds(i * bm, bm), :]
    old_rows = embed_ref[pl.ds(buf * n_nodes + i * bm, bm), :]
    new_embed = jnp.maximum(merged, 0.0) + old_rows

    @pl.when(lv < MAX_LEVELS - 1)
    def _update():
        embed_ref[pl.ds((1 - buf) * n_nodes + i * bm, bm), :] = new_embed

    @pl.when(lv == MAX_LEVELS - 1)
    def _head():
        logits = jnp.dot(new_embed, last_w_ref[...], precision=_PRECISION)
        logits = logits + last_b_ref[...]
        m = jnp.max(logits, axis=1, keepdims=True)
        shifted = logits - m
        lse = jnp.log(jnp.sum(jnp.exp(shifted), axis=1, keepdims=True))
        ls = shifted - lse
        col = jax.lax.broadcasted_iota(jnp.int32, ls.shape, 1)
        is_max = ls >= jnp.max(ls, axis=1, keepdims=True)
        pred = jnp.min(jnp.where(is_max, col, n_classes), axis=1)
        pred_ref[0, 0, :] = pred
        labels = labels_ref[0, 0, :]
        acc_ref[0, 0, :] = (pred == labels).astype(jnp.int32)
        sel = jnp.sum(jnp.where(col == labels[:, None], ls, 0.0), axis=1)
        part_loss = jnp.sum(sel)
        total = jnp.where(i == 0, part_loss, lsum_ref[0] + part_loss)
        lsum_ref[0] = total

        @pl.when(i == n_blocks - 1)
        def _emit_loss():
            loss_ref[...] = jnp.full((1, 1), -total / n_nodes, jnp.float32)


def kernel(x, adj, labels, w_n2l, b_n2l, conv_w, conv_b, last_w, last_b):
    n, f_dim = x.shape
    l_dim = conv_w.shape[0]
    n_classes = last_w.shape[1]
    bm = _pick(n, (200, 256, 128, 100, 80, 50, 40, 25, 16, 10, 8, 5, 4, 2, 1))
    rb = _pick(n, (1000, 800, 500, 400, 200, 100, 50, 25, 10, 8, 5, 4, 2, 1))
    ni = n // bm

    labels3 = labels.astype(jnp.int32).reshape(ni, 1, bm)
    b_n2l2 = b_n2l.reshape(1, l_dim)
    conv_b2 = conv_b.reshape(1, l_dim)
    last_b2 = last_b.reshape(1, n_classes)

    body = functools.partial(_body, bm=bm, rb=rb, n_nodes=n,
                             n_classes=n_classes, n_blocks=ni)
    pred3, loss2, acc3 = pl.pallas_call(
        body,
        grid=(MAX_LEVELS, ni),
        in_specs=[
            pl.BlockSpec((bm, n), lambda lv, i: (i, 0)),
            pl.BlockSpec((n, f_dim), lambda lv, i: (0, 0)),
            pl.BlockSpec((1, 1, bm), lambda lv, i: (i, 0, 0)),
            pl.BlockSpec((f_dim, l_dim), lambda lv, i: (0, 0)),
            pl.BlockSpec((1, l_dim), lambda lv, i: (0, 0)),
            pl.BlockSpec((l_dim, l_dim), lambda lv, i: (0, 0)),
            pl.BlockSpec((1, l_dim), lambda lv, i: (0, 0)),
            pl.BlockSpec((l_dim, n_classes), lambda lv, i: (0, 0)),
            pl.BlockSpec((1, n_classes), lambda lv, i: (0, 0)),
        ],
        out_specs=(
            pl.BlockSpec((1, 1, bm), lambda lv, i: (i, 0, 0)),
            pl.BlockSpec((1, 1), lambda lv, i: (0, 0)),
            pl.BlockSpec((1, 1, bm), lambda lv, i: (i, 0, 0)),
        ),
        out_shape=(
            jax.ShapeDtypeStruct((ni, 1, bm), jnp.int32),
            jax.ShapeDtypeStruct((1, 1), jnp.float32),
            jax.ShapeDtypeStruct((ni, 1, bm), jnp.int32),
        ),
        scratch_shapes=[
            pltpu.VMEM((2 * n, l_dim), jnp.float32),
            pltpu.VMEM((n, l_dim), jnp.float32),
            pltpu.SMEM((1,), jnp.float32),
        ],
        compiler_params=pltpu.CompilerParams(
            dimension_semantics=("arbitrary", "arbitrary"),
        ),
    )(adj, x, labels3, w_n2l, b_n2l2, conv_w, conv_b2, last_w, last_b2)

    pred = pred3.reshape(n, 1)
    loss = loss2[0, 0]
    acc = acc3.reshape(n) != 0
    return pred, loss, acc


# speedup vs baseline: 2.8986x; 1.0895x over previous
import functools

import jax
import jax.numpy as jnp
from jax.experimental import pallas as pl
from jax.experimental.pallas import tpu as pltpu

MAX_LEVELS = 3
_KCHUNK = 2048
_PRECISION = jax.lax.Precision.DEFAULT
_ADJ_PRECISION = jax.lax.Precision.DEFAULT


def _pick(n, cands):
    for c in cands:
        if n % c == 0:
            return c
    return n


def _kchunks(n):
    offs, sizes = [], []
    o = 0
    while o < n:
        sz = min(_KCHUNK, n - o)
        offs.append(o)
        sizes.append(sz)
        o += sz
    return tuple(zip(offs, sizes))


def _body(adj_ref, x_ref, labels_ref, w_n2l_ref, b_n2l_ref, conv_w_ref,
          conv_b_ref, last_w_ref, last_b_ref,
          pred_ref, loss_ref, acc_ref,
          embed_ref, in_lin_ref, lsum_ref,
          *, bm, rb, n_nodes, n_classes, n_blocks):
    lv = pl.program_id(0)
    i = pl.program_id(1)

    @pl.when(jnp.logical_and(lv == 0, i == 0))
    def _init():
        def chunk(r, carry):
            xa = x_ref[pl.ds(r * rb, rb), :]
            il = jnp.dot(xa, w_n2l_ref[...], precision=_PRECISION)
            il = il + b_n2l_ref[...]
            in_lin_ref[pl.ds(r * rb, rb), :] = il
            embed_ref[pl.ds(r * rb, rb), :] = jnp.maximum(il, 0.0)
            return carry
        jax.lax.fori_loop(0, n_nodes // rb, chunk, 0)

    buf = lv % 2
    n2npool = None
    for off, sz in _kchunks(n_nodes):
        e_chunk = embed_ref[pl.ds(buf * n_nodes + off, sz), :]
        part = jnp.dot(adj_ref[:, off:off + sz], e_chunk,
                       precision=_ADJ_PRECISION)
        n2npool = part if n2npool is None else n2npool + part

    node_linear = jnp.dot(n2npool, conv_w_ref[...], precision=_PRECISION)
    merged = node_linear + conv_b_ref[...] + in_lin_ref[pl.ds(i * bm, bm), :]
    old_rows = embed_ref[pl.ds(buf * n_nodes + i * bm, bm), :]
    new_embed = jnp.maximum(merged, 0.0) + old_rows

    @pl.when(lv < MAX_LEVELS - 1)
    def _update():
        embed_ref[pl.ds((1 - buf) * n_nodes + i * bm, bm), :] = new_embed

    @pl.when(lv == MAX_LEVELS - 1)
    def _head():
        logits = jnp.dot(new_embed, last_w_ref[...], precision=_PRECISION)
        logits = logits + last_b_ref[...]
        m = jnp.max(logits, axis=1, keepdims=True)
        shifted = logits - m
        lse = jnp.log(jnp.sum(jnp.exp(shifted), axis=1, keepdims=True))
        ls = shifted - lse
        col = jax.lax.broadcasted_iota(jnp.int32, ls.shape, 1)
        is_max = ls >= jnp.max(ls, axis=1, keepdims=True)
        pred = jnp.min(jnp.where(is_max, col, n_classes), axis=1)
        pred_ref[0, 0, :] = pred
        labels = labels_ref[0, 0, :]
        acc_ref[0, 0, :] = (pred == labels).astype(jnp.int32)
        sel = jnp.sum(jnp.where(col == labels[:, None], ls, 0.0), axis=1)
        part_loss = jnp.sum(sel)
        total = jnp.where(i == 0, part_loss, lsum_ref[0] + part_loss)
        lsum_ref[0] = total

        @pl.when(i == n_blocks - 1)
        def _emit_loss():
            loss_ref[...] = jnp.full((1, 1), -total / n_nodes, jnp.float32)


def kernel(x, adj, labels, w_n2l, b_n2l, conv_w, conv_b, last_w, last_b):
    n, f_dim = x.shape
    l_dim = conv_w.shape[0]
    n_classes = last_w.shape[1]
    bm = _pick(n, (200, 256, 128, 100, 80, 50, 40, 25, 16, 10, 8, 5, 4, 2, 1))
    rb = _pick(n, (1000, 800, 500, 400, 200, 100, 50, 25, 10, 8, 5, 4, 2, 1))
    ni = n // bm

    labels3 = labels.astype(jnp.int32).reshape(ni, 1, bm)
    b_n2l2 = b_n2l.reshape(1, l_dim)
    conv_b2 = conv_b.reshape(1, l_dim)
    last_b2 = last_b.reshape(1, n_classes)

    body = functools.partial(_body, bm=bm, rb=rb, n_nodes=n,
                             n_classes=n_classes, n_blocks=ni)
    pred3, loss2, acc3 = pl.pallas_call(
        body,
        grid=(MAX_LEVELS, ni),
        in_specs=[
            pl.BlockSpec((bm, n), lambda lv, i: (i, 0)),
            pl.BlockSpec((n, f_dim), lambda lv, i: (0, 0)),
            pl.BlockSpec((1, 1, bm), lambda lv, i: (i, 0, 0)),
            pl.BlockSpec((f_dim, l_dim), lambda lv, i: (0, 0)),
            pl.BlockSpec((1, l_dim), lambda lv, i: (0, 0)),
            pl.BlockSpec((l_dim, l_dim), lambda lv, i: (0, 0)),
            pl.BlockSpec((1, l_dim), lambda lv, i: (0, 0)),
            pl.BlockSpec((l_dim, n_classes), lambda lv, i: (0, 0)),
            pl.BlockSpec((1, n_classes), lambda lv, i: (0, 0)),
        ],
        out_specs=(
            pl.BlockSpec((1, 1, bm), lambda lv, i: (i, 0, 0)),
            pl.BlockSpec((1, 1), lambda lv, i: (0, 0)),
            pl.BlockSpec((1, 1, bm), lambda lv, i: (i, 0, 0)),
        ),
        out_shape=(
            jax.ShapeDtypeStruct((ni, 1, bm), jnp.int32),
            jax.ShapeDtypeStruct((1, 1), jnp.float32),
            jax.ShapeDtypeStruct((ni, 1, bm), jnp.int32),
        ),
        scratch_shapes=[
            pltpu.VMEM((2 * n, l_dim), jnp.float32),
            pltpu.VMEM((n, l_dim), jnp.float32),
            pltpu.SMEM((1,), jnp.float32),
        ],
        compiler_params=pltpu.CompilerParams(
            dimension_semantics=("arbitrary", "arbitrary"),
        ),
    )(adj, x, labels3, w_n2l, b_n2l2, conv_w, conv_b2, last_w, last_b2)

    pred = pred3.reshape(n, 1)
    loss = loss2[0, 0]
    acc = acc3.reshape(n) != 0
    return pred, loss, acc
